# baseline (speedup 1.0000x reference)
import functools
import jax
import jax.numpy as jnp
from jax import lax
from jax.experimental import pallas as pl
from jax.experimental.pallas import tpu as pltpu
from jax.experimental.pallas import tpu_sc as plsc

N, M, D = 10000, 32, 128
L = 16
NW = 32
NPW = N // NW
REM = N - NPW * NW
K = 4
NCH = NPW // K
ROWS = K * M


def _sc_body(x_hbm, o_hbm, buf0, buf1, ob, sem0, sem1):
    c = lax.axis_index("c")
    s = lax.axis_index("s")
    wid = s * 2 + c
    base_node = wid * NPW
    base_row = base_node * M

    def copy_in(g, buf, sem):
        return pltpu.make_async_copy(
            x_hbm.at[pl.ds(base_row + g * ROWS, ROWS)], buf, sem)

    sls = [pl.ds(dc * L, L) for dc in range(D // L)]
    NDC = D // L

    def reduce_chunk(buf, g):
        def mstep(m, accs):
            return tuple(
                accs[k * NDC + dc] + buf[k * M + m, sls[dc]]
                for k in range(K) for dc in range(NDC))

        init = tuple(
            buf[k * M, sls[dc]] for k in range(K) for dc in range(NDC))
        accs = lax.fori_loop(1, M, mstep, init)
        for k in range(K):
            for dc in range(NDC):
                ob[k, sls[dc]] = accs[k * NDC + dc]
        pltpu.sync_copy(ob, o_hbm.at[pl.ds(base_node + g * K, K)])

    copy_in(0, buf0, sem0).start()
    copy_in(1, buf1, sem1).start()

    def step(i, carry):
        for b, (buf, sem) in enumerate(((buf0, sem0), (buf1, sem1))):
            g = i * 2 + b
            copy_in(g, buf, sem).wait()
            reduce_chunk(buf, g)
            nxt = g + 2

            @pl.when(nxt < NCH)
            def _(buf=buf, sem=sem, nxt=nxt):
                copy_in(nxt, buf, sem).start()
        return carry

    lax.fori_loop(0, NCH // 2, step, 0)

    @pl.when(wid < REM)
    def _():
        tail_node = N - REM + wid
        pltpu.sync_copy(x_hbm.at[pl.ds(tail_node * M, M)],
                        buf0.at[pl.ds(0, M)])
        def mstep(m, accs):
            return tuple(accs[dc] + buf0[m, sls[dc]] for dc in range(NDC))

        accs = lax.fori_loop(
            1, M, mstep, tuple(buf0[0, sls[dc]] for dc in range(NDC)))
        for dc in range(NDC):
            ob[0, sls[dc]] = accs[dc]
        pltpu.sync_copy(ob.at[pl.ds(0, 1)], o_hbm.at[pl.ds(tail_node, 1)])


_sc_call = functools.partial(
    pl.kernel,
    out_type=jax.ShapeDtypeStruct((N, D), jnp.float32),
    mesh=plsc.VectorSubcoreMesh(core_axis_name="c", subcore_axis_name="s"),
    scratch_types=[
        pltpu.VMEM((ROWS, D), jnp.float32),
        pltpu.VMEM((ROWS, D), jnp.float32),
        pltpu.VMEM((K, D), jnp.float32),
        pltpu.SemaphoreType.DMA,
        pltpu.SemaphoreType.DMA,
    ],
)(_sc_body)


def kernel(messages):
    x = messages.reshape(N * M, D)
    out = _sc_call(x)
    return out.reshape(1, N, D)


# speedup vs baseline: 1.4225x; 1.4225x over previous
import functools
import jax
import jax.numpy as jnp
from jax import lax
from jax.experimental import pallas as pl
from jax.experimental.pallas import tpu as pltpu
from jax.experimental.pallas import tpu_sc as plsc

N, M, D = 10000, 32, 128
L = 16
NW = 32
NDC = D // L

S_SC = 3200
NPW = S_SC // NW
K = 2
NCH = NPW // K
ROWS = K * M

NB = 200
TC_OFF = S_SC // NB


def _sc_body(x_hbm, o_hbm, buf0, buf1, ob, sem0, sem1):
    c = lax.axis_index("c")
    s = lax.axis_index("s")
    wid = s * 2 + c
    base_node = wid * NPW
    base_row = base_node * M

    def copy_in(g, buf, sem):
        return pltpu.make_async_copy(
            x_hbm.at[pl.ds(base_row + g * ROWS, ROWS)], buf, sem)

    sls = [pl.ds(dc * L, L) for dc in range(NDC)]

    def reduce_chunk(buf, g):
        def mstep(m, accs):
            return tuple(
                accs[k * NDC + dc] + buf[k * M + m, sls[dc]]
                for k in range(K) for dc in range(NDC))

        init = tuple(
            buf[k * M, sls[dc]] for k in range(K) for dc in range(NDC))
        accs = lax.fori_loop(1, M, mstep, init)
        for k in range(K):
            for dc in range(NDC):
                ob[k, sls[dc]] = accs[k * NDC + dc]
        pltpu.sync_copy(ob, o_hbm.at[pl.ds(base_node + g * K, K)])

    copy_in(0, buf0, sem0).start()
    copy_in(1, buf1, sem1).start()

    def step(i, carry):
        for b, (buf, sem) in enumerate(((buf0, sem0), (buf1, sem1))):
            g = i * 2 + b
            copy_in(g, buf, sem).wait()
            reduce_chunk(buf, g)
            nxt = g + 2

            @pl.when(nxt < NCH)
            def _(buf=buf, sem=sem, nxt=nxt):
                copy_in(nxt, buf, sem).start()
        return carry

    lax.fori_loop(0, NCH // 2, step, 0)


_sc_call = functools.partial(
    pl.kernel,
    out_type=jax.ShapeDtypeStruct((S_SC, D), jnp.float32),
    mesh=plsc.VectorSubcoreMesh(core_axis_name="c", subcore_axis_name="s"),
    scratch_types=[
        pltpu.VMEM((ROWS, D), jnp.float32),
        pltpu.VMEM((ROWS, D), jnp.float32),
        pltpu.VMEM((K, D), jnp.float32),
        pltpu.SemaphoreType.DMA,
        pltpu.SemaphoreType.DMA,
    ],
)(_sc_body)


def _tc_body(x_ref, o_ref):
    o_ref[...] = jnp.sum(x_ref[...], axis=1)


def _tc_call(x3):
    return pl.pallas_call(
        _tc_body,
        grid=((N - S_SC) // NB,),
        in_specs=[pl.BlockSpec((NB, M, D), lambda i: (i + TC_OFF, 0, 0))],
        out_specs=pl.BlockSpec((NB, D), lambda i: (i, 0)),
        out_shape=jax.ShapeDtypeStruct((N - S_SC, D), jnp.float32),
    )(x3)


def kernel(messages):
    x3 = messages.reshape(N, M, D)
    sc_out = _sc_call(x3.reshape(N * M, D))
    tc_out = _tc_call(x3)
    out = jnp.concatenate([sc_out, tc_out], axis=0)
    return out.reshape(1, N, D)
